# baseline (speedup 1.0000x reference)
import functools

import jax
import jax.numpy as jnp
from jax.experimental import pallas as pl
from jax.experimental.pallas import tpu as pltpu

_TI = 512


def _moe_body(idx_ref, w_ref, x_ref, g_ref, u_ref, d_ref, out_ref,
              xt_scr, acc_scr):
    e = pl.program_id(0)
    i = pl.program_id(1)
    first = jnp.logical_and(e == 0, i == 0)
    last = jnp.logical_and(e == pl.num_programs(0) - 1,
                           i == pl.num_programs(1) - 1)

    @pl.when(first)
    def _init():
        xt_scr[...] = x_ref[...].T
        acc_scr[...] = jnp.zeros_like(acc_scr)

    xt = xt_scr[...]
    g = jax.lax.dot_general(g_ref[0], xt, (((1,), (0,)), ((), ())),
                            preferred_element_type=jnp.float32)
    u = jax.lax.dot_general(u_ref[0], xt, (((1,), (0,)), ((), ())),
                            preferred_element_type=jnp.float32)
    h = (g * jax.nn.sigmoid(g)) * u

    ce = jnp.sum(jnp.where(idx_ref[...] == e, w_ref[...], 0.0), axis=1)
    h = h * ce[None, :]

    acc_scr[...] += jax.lax.dot_general(d_ref[0], h, (((1,), (0,)), ((), ())),
                                        preferred_element_type=jnp.float32)

    @pl.when(last)
    def _emit():
        out_ref[...] = acc_scr[...].T


@functools.partial(jax.jit, static_argnames=())
def kernel(x, expert_indices, expert_weights, gate_proj, up_proj, down_proj):
    batch, seq_len, hidden = x.shape
    num_experts = gate_proj.shape[0]
    inter = gate_proj.shape[1]
    top_k = expert_indices.shape[-1]
    num_tokens = batch * seq_len

    x2 = x.reshape(num_tokens, hidden)
    idx = expert_indices.reshape(num_tokens, top_k)
    w = expert_weights.reshape(num_tokens, top_k)

    n_i = inter // _TI
    grid = (num_experts, n_i)

    out = pl.pallas_call(
        _moe_body,
        grid=grid,
        in_specs=[
            pl.BlockSpec((num_tokens, top_k), lambda e, i: (0, 0)),
            pl.BlockSpec((num_tokens, top_k), lambda e, i: (0, 0)),
            pl.BlockSpec((num_tokens, hidden), lambda e, i: (0, 0)),
            pl.BlockSpec((1, _TI, hidden), lambda e, i: (e, i, 0)),
            pl.BlockSpec((1, _TI, hidden), lambda e, i: (e, i, 0)),
            pl.BlockSpec((1, hidden, _TI), lambda e, i: (e, 0, i)),
        ],
        out_specs=pl.BlockSpec((num_tokens, hidden), lambda e, i: (0, 0)),
        out_shape=jax.ShapeDtypeStruct((num_tokens, hidden), jnp.float32),
        scratch_shapes=[
            pltpu.VMEM((hidden, num_tokens), jnp.float32),
            pltpu.VMEM((hidden, num_tokens), jnp.float32),
        ],
    )(idx, w, x2, gate_proj, up_proj, down_proj)

    return out.reshape(batch, seq_len, hidden)
